# Initial kernel scaffold; baseline (speedup 1.0000x reference)
import jax
import jax.numpy as jnp
from jax.experimental import pallas as pl


def kernel(user_embed, item_embed, edge_row, edge_col, edge_val):
    raise NotImplementedError("write your pallas kernel here")


# speedup vs baseline: 1.7724x; 1.7724x over previous
import functools

import jax
import jax.numpy as jnp
from jax import lax
from jax.experimental import pallas as pl
from jax.experimental.pallas import tpu as pltpu
from jax.experimental.pallas import tpu_sc as plsc

_N_USERS = 5000
_N_ITEMS = 5000
_N_NODES = _N_USERS + _N_ITEMS
_N_EDGES = 320000
_D = 128
_DH = _D // 2
_N_HOPS = 3

_NS = 16
_EPT = _N_EDGES // _NS
_K = 80
_NCHUNK = _EPT // _K
_NPAD = 10240
_RPT = _NPAD // _NS
_ZR = 128


def _hop_body(scale, ego_a, ego_b, sum_a, sum_b, row_hbm, col_hbm, val_hbm,
              out_a, out_b, sout_a, sout_b,
              acc, col_v, row_v, val_v, rows_v, zbuf, abuf, sbuf, sem):
    c = lax.axis_index("c")
    s = lax.axis_index("s")

    def zrow(i, carry):
        for d in range(_DH // 16):
            zbuf[i, pl.ds(16 * d, 16)] = jnp.zeros((16,), jnp.float32)
        return carry

    lax.fori_loop(0, _ZR, zrow, 0)
    for j in range(_RPT // _ZR):
        r0 = pl.multiple_of(_RPT * s + _ZR * j, _ZR)
        pltpu.sync_copy(zbuf, acc.at[pl.ds(r0, _ZR)])
    plsc.subcore_barrier()

    def chunk(i, carry):
        base = pl.multiple_of(s * _EPT + i * _K, _K)
        pltpu.sync_copy(col_hbm.at[pl.ds(base, _K)], col_v)
        pltpu.sync_copy(row_hbm.at[pl.ds(base, _K)], row_v)
        pltpu.sync_copy(val_hbm.at[pl.ds(base, _K)], val_v)

        @pl.when(c == 0)
        def _():
            pltpu.async_copy(ego_a.at[col_v], rows_v, sem).wait()

        @pl.when(c == 1)
        def _():
            pltpu.async_copy(ego_b.at[col_v], rows_v, sem).wait()

        def scale_group(g, carry2):
            vv = val_v[pl.ds(16 * g, 16)]
            for j in range(16):
                v = vv[j]
                e = 16 * g + j
                for d in range(_DH // 16):
                    sl = pl.ds(16 * d, 16)
                    rows_v[e, sl] = rows_v[e, sl] * v
            return carry2

        lax.fori_loop(0, _K // 16, scale_group, 0)
        pltpu.sync_copy(rows_v, acc.at[row_v], add=True)
        return carry

    lax.fori_loop(0, _NCHUNK, chunk, 0)
    plsc.subcore_barrier()

    for cc, ego_out, sum_in, sum_out in (
        (0, out_a, sum_a, sout_a),
        (1, out_b, sum_b, sout_b),
    ):
        @pl.when(c == cc)
        def _(ego_out=ego_out, sum_in=sum_in, sum_out=sum_out):
            for j in range(_RPT // _ZR):
                r0 = pl.multiple_of(_RPT * s + _ZR * j, _ZR)
                rows = pl.ds(r0, _ZR)
                pltpu.sync_copy(acc.at[rows], ego_out.at[rows])
                pltpu.sync_copy(acc.at[rows], abuf)
                pltpu.sync_copy(sum_in.at[rows], sbuf)

                def addrow(i, carry):
                    for d in range(_DH // 16):
                        sl = pl.ds(16 * d, 16)
                        sbuf[i, sl] = (sbuf[i, sl] + abuf[i, sl]) * scale
                    return carry

                lax.fori_loop(0, _ZR, addrow, 0)
                pltpu.sync_copy(sbuf, sum_out.at[rows])


@functools.lru_cache(maxsize=None)
def _make_hop(scale):
    half = jax.ShapeDtypeStruct((_NPAD, _DH), jnp.float32)
    return pl.kernel(
        functools.partial(_hop_body, scale),
        out_type=(half, half, half, half),
        mesh=plsc.VectorSubcoreMesh(core_axis_name="c", subcore_axis_name="s"),
        scratch_types=[
            pltpu.VMEM_SHARED((_NPAD, _DH), jnp.float32),
            pltpu.VMEM((_K,), jnp.int32),
            pltpu.VMEM((_K,), jnp.int32),
            pltpu.VMEM((_K,), jnp.float32),
            pltpu.VMEM((_K, _DH), jnp.float32),
            pltpu.VMEM((_ZR, _DH), jnp.float32),
            pltpu.VMEM((_ZR, _DH), jnp.float32),
            pltpu.VMEM((_ZR, _DH), jnp.float32),
            pltpu.SemaphoreType.DMA,
        ],
        compiler_params=pltpu.CompilerParams(use_tc_tiling_on_sc=False),
        name=f"gcn_hop_scale_{scale}",
    )


def kernel(user_embed, item_embed, edge_row, edge_col, edge_val):
    ego = jnp.concatenate([user_embed, item_embed], axis=0)
    pad = jnp.zeros((_NPAD - _N_NODES, _DH), jnp.float32)
    ego_a = jnp.concatenate([ego[:, :_DH], pad], axis=0)
    ego_b = jnp.concatenate([ego[:, _DH:], pad], axis=0)
    row = edge_row.astype(jnp.int32)
    col = edge_col.astype(jnp.int32)
    val = edge_val.astype(jnp.float32)
    sum_a = jnp.zeros((_NPAD, _DH), jnp.float32)
    sum_b = jnp.zeros((_NPAD, _DH), jnp.float32)
    for h in range(_N_HOPS):
        scale = (1.0 / _N_HOPS) if h == _N_HOPS - 1 else 1.0
        hop = _make_hop(scale)
        ego_a, ego_b, sum_a, sum_b = hop(
            ego_a, ego_b, sum_a, sum_b, row, col, val)
    all_emb = jnp.concatenate([sum_a[:_N_NODES], sum_b[:_N_NODES]], axis=1)
    return all_emb[:_N_USERS], all_emb[_N_USERS:]


# speedup vs baseline: 3.5124x; 1.9817x over previous
import functools

import jax
import jax.numpy as jnp
from jax import lax
from jax.experimental import pallas as pl
from jax.experimental.pallas import tpu as pltpu
from jax.experimental.pallas import tpu_sc as plsc

_N_USERS = 5000
_N_ITEMS = 5000
_N_NODES = _N_USERS + _N_ITEMS
_N_EDGES = 320000
_D = 128
_DH = _D // 2
_N_HOPS = 3

_NS = 16
_K = 128
_NCHUNK = 160
_EPT = _NCHUNK * _K
_NEPAD = _NS * _EPT
_NPAD = 10240
_RPT = _NPAD // _NS
_ZR = 64
_DUMMY_ROW = _N_NODES + 100
_NB = 4
_VG = 8


def _hop_body(scale, ego_hbm, sum_hbm, row_hbm, col0_hbm, col1_hbm, val_hbm,
              out_hbm, sout_hbm,
              acc, col_all, row_all, vbuf, rv0, rv1, rv2, rv3, abuf, sbuf,
              g0, g1, g2, g3, s0, s1, s2, s3):
    c = lax.axis_index("c")
    s = lax.axis_index("s")
    rvs = (rv0, rv1, rv2, rv3)
    gsems = (g0, g1, g2, g3)
    ssems = (s0, s1, s2, s3)

    tb = pl.multiple_of(s * _NCHUNK, _NCHUNK)
    trows = pl.ds(tb, _NCHUNK)

    @pl.when(c == 0)
    def _():
        pltpu.sync_copy(col0_hbm.at[trows], col_all)

    @pl.when(c == 1)
    def _():
        pltpu.sync_copy(col1_hbm.at[trows], col_all)

    pltpu.sync_copy(row_hbm.at[trows], row_all)

    def zrow(i, carry):
        for d in range(_DH // 16):
            abuf[i, pl.ds(16 * d, 16)] = jnp.zeros((16,), jnp.float32)
        return carry

    lax.fori_loop(0, _ZR, zrow, 0)
    for j in range(_RPT // _ZR):
        r0 = pl.multiple_of(_RPT * s + _ZR * j, _ZR)
        pltpu.sync_copy(abuf, acc.at[pl.ds(r0, _ZR)])
    plsc.subcore_barrier()

    def issue_gather(i, b):
        pltpu.async_copy(ego_hbm.at[col_all.at[i]], rvs[b], gsems[b])

    def wait_gather(b):
        pltpu.make_async_copy(ego_hbm.at[pl.ds(0, _K)], rvs[b],
                              gsems[b]).wait()

    def issue_scatter(i, b):
        pltpu.async_copy(rvs[b], acc.at[row_all.at[i]], ssems[b], add=True)

    def wait_scatter(b):
        pltpu.make_async_copy(rvs[b], acc.at[pl.ds(0, _K)], ssems[b]).wait()

    def scale_buf(j, b):
        rv = rvs[b]

        def sg(g, carry):
            vv = vbuf[j, pl.ds(16 * g, 16)]
            for jj in range(16):
                v = vv[jj]
                e = 16 * g + jj
                for d in range(_DH // 16):
                    sl = pl.ds(16 * d, 16)
                    rv[e, sl] = rv[e, sl] * v
            return carry

        lax.fori_loop(0, _K // 16, sg, 0)

    issue_gather(0, 0)
    issue_gather(1, 1)

    def super_group(k, carry):
        i0 = _VG * k
        vb = pl.multiple_of(s * _NCHUNK + i0, _VG)
        pltpu.sync_copy(val_hbm.at[pl.ds(vb, _VG)], vbuf)
        for j in range(_VG):
            i = i0 + j
            b = j % _NB
            bn = (j + 2) % _NB

            @pl.when(i >= 2)
            def _():
                wait_scatter(bn)

            @pl.when(i + 2 < _NCHUNK)
            def _():
                issue_gather(i + 2, bn)

            wait_gather(b)
            scale_buf(j, b)
            issue_scatter(i, b)
        return carry

    lax.fori_loop(0, _NCHUNK // _VG, super_group, 0)
    for b in ((_NCHUNK - 2) % _NB, (_NCHUNK - 1) % _NB):
        wait_scatter(b)
    plsc.subcore_barrier()

    for j in range(_RPT // _ZR):
        r0 = pl.multiple_of(_RPT * s + _ZR * j, _ZR)
        ro = pl.multiple_of(c * _NPAD + _RPT * s + _ZR * j, _ZR)
        lrows = pl.ds(r0, _ZR)
        hrows = pl.ds(ro, _ZR)
        pltpu.sync_copy(acc.at[lrows], out_hbm.at[hrows])
        pltpu.sync_copy(acc.at[lrows], abuf)
        pltpu.sync_copy(sum_hbm.at[hrows], sbuf)

        def addrow(i, carry):
            for d in range(_DH // 16):
                sl = pl.ds(16 * d, 16)
                sbuf[i, sl] = (sbuf[i, sl] + abuf[i, sl]) * scale
            return carry

        lax.fori_loop(0, _ZR, addrow, 0)
        pltpu.sync_copy(sbuf, sout_hbm.at[hrows])


@functools.lru_cache(maxsize=None)
def _make_hop(scale):
    full = jax.ShapeDtypeStruct((2 * _NPAD, _DH), jnp.float32)
    return pl.kernel(
        functools.partial(_hop_body, scale),
        out_type=(full, full),
        mesh=plsc.VectorSubcoreMesh(core_axis_name="c", subcore_axis_name="s"),
        scratch_types=[
            pltpu.VMEM_SHARED((_NPAD, _DH), jnp.float32),
            pltpu.VMEM((_NCHUNK, _K), jnp.int32),
            pltpu.VMEM((_NCHUNK, _K), jnp.int32),
            pltpu.VMEM((_VG, _K), jnp.float32),
            pltpu.VMEM((_K, _DH), jnp.float32),
            pltpu.VMEM((_K, _DH), jnp.float32),
            pltpu.VMEM((_K, _DH), jnp.float32),
            pltpu.VMEM((_K, _DH), jnp.float32),
            pltpu.VMEM((_ZR, _DH), jnp.float32),
            pltpu.VMEM((_ZR, _DH), jnp.float32),
            pltpu.SemaphoreType.DMA,
            pltpu.SemaphoreType.DMA,
            pltpu.SemaphoreType.DMA,
            pltpu.SemaphoreType.DMA,
            pltpu.SemaphoreType.DMA,
            pltpu.SemaphoreType.DMA,
            pltpu.SemaphoreType.DMA,
            pltpu.SemaphoreType.DMA,
        ],
        compiler_params=pltpu.CompilerParams(use_tc_tiling_on_sc=False),
        name=f"gcn_hop_scale_{scale}",
    )


def kernel(user_embed, item_embed, edge_row, edge_col, edge_val):
    ego = jnp.concatenate([user_embed, item_embed], axis=0)
    pad = jnp.zeros((_NPAD - _N_NODES, _DH), jnp.float32)
    ego_ab = jnp.concatenate([ego[:, :_DH], pad, ego[:, _DH:], pad], axis=0)

    epad = _NEPAD - _N_EDGES
    row = jnp.concatenate([
        edge_row.astype(jnp.int32),
        jnp.full((epad,), _DUMMY_ROW, jnp.int32),
    ]).reshape(_NS * _NCHUNK, _K)
    col = jnp.concatenate([
        edge_col.astype(jnp.int32),
        jnp.zeros((epad,), jnp.int32),
    ])
    col0 = col.reshape(_NS * _NCHUNK, _K)
    col1 = (col + _NPAD).reshape(_NS * _NCHUNK, _K)
    val = jnp.concatenate([
        edge_val.astype(jnp.float32),
        jnp.zeros((epad,), jnp.float32),
    ]).reshape(_NS * _NCHUNK, _K)

    sum_ab = jnp.zeros((2 * _NPAD, _DH), jnp.float32)
    for h in range(_N_HOPS):
        scale = (1.0 / _N_HOPS) if h == _N_HOPS - 1 else 1.0
        hop = _make_hop(scale)
        ego_ab, sum_ab = hop(ego_ab, sum_ab, row, col0, col1, val)

    all_emb = jnp.concatenate(
        [sum_ab[:_N_NODES], sum_ab[_NPAD:_NPAD + _N_NODES]], axis=1)
    return all_emb[:_N_USERS], all_emb[_N_USERS:]


# speedup vs baseline: 4.8891x; 1.3920x over previous
import functools

import jax
import jax.numpy as jnp
from jax import lax
from jax.experimental import pallas as pl
from jax.experimental.pallas import tpu as pltpu
from jax.experimental.pallas import tpu_sc as plsc

_N_USERS = 5000
_N_ITEMS = 5000
_N_NODES = _N_USERS + _N_ITEMS
_N_EDGES = 320000
_D = 128
_DH = _D // 2
_N_HOPS = 3

_NS = 16
_K = 128
_NCHUNK = 160
_EPT = _NCHUNK * _K
_NEPAD = _NS * _EPT
_NPAD = 10240
_RPT = _NPAD // _NS
_ZR = 64
_DUMMY_ROW = _N_NODES + 100
_NB = 4
_VG = 8


def _hop_body(scale, ego_hbm, sum_hbm, row_hbm, col0_hbm, col1_hbm, val_hbm,
              out_hbm, sout_hbm,
              acc, col3, row3, val3, ri0, ri1, ri2, ri3, ro0, ro1, ro2, ro3,
              abuf, sbuf, obuf,
              g0, g1, g2, g3, s0, s1, s2, s3, esem):
    c = lax.axis_index("c")
    s = lax.axis_index("s")
    rins = (ri0, ri1, ri2, ri3)
    routs = (ro0, ro1, ro2, ro3)
    gsems = (g0, g1, g2, g3)
    ssems = (s0, s1, s2, s3)
    _NG = _NCHUNK // _VG

    def edata_rows(g):
        eb = pl.multiple_of(s * _NCHUNK + _VG * g, _VG)
        return pl.ds(eb, _VG)

    def issue_edata(g):
        slot = lax.rem(g, 3)
        rows = edata_rows(g)

        @pl.when(c == 0)
        def _():
            pltpu.async_copy(col0_hbm.at[rows], col3.at[slot], esem)

        @pl.when(c == 1)
        def _():
            pltpu.async_copy(col1_hbm.at[rows], col3.at[slot], esem)

        pltpu.async_copy(row_hbm.at[rows], row3.at[slot], esem)
        pltpu.async_copy(val_hbm.at[rows], val3.at[slot], esem)

    def wait_edata():
        pltpu.make_async_copy(row_hbm.at[edata_rows(0)], col3.at[0],
                              esem).wait()
        pltpu.make_async_copy(row_hbm.at[edata_rows(0)], row3.at[0],
                              esem).wait()
        pltpu.make_async_copy(val_hbm.at[edata_rows(0)], val3.at[0],
                              esem).wait()

    slot0 = lax.rem(0 * s, 3)

    @pl.when(c == 0)
    def _():
        pltpu.sync_copy(col0_hbm.at[edata_rows(0)], col3.at[slot0])

    @pl.when(c == 1)
    def _():
        pltpu.sync_copy(col1_hbm.at[edata_rows(0)], col3.at[slot0])

    pltpu.sync_copy(row_hbm.at[edata_rows(0)], row3.at[slot0])
    pltpu.sync_copy(val_hbm.at[edata_rows(0)], val3.at[slot0])
    issue_edata(1)

    def zrow(i, carry):
        for d in range(_DH // 16):
            abuf[i, pl.ds(16 * d, 16)] = jnp.zeros((16,), jnp.float32)
        return carry

    lax.fori_loop(0, _ZR, zrow, 0)
    for j in range(_RPT // _ZR):
        r0 = pl.multiple_of(_RPT * s + _ZR * j, _ZR)
        pltpu.sync_copy(abuf, acc.at[pl.ds(r0, _ZR)])
    plsc.subcore_barrier()

    def issue_gather(slot, jj, b):
        pltpu.async_copy(ego_hbm.at[col3.at[slot, jj]], rins[b], gsems[b])

    def wait_gather(b):
        pltpu.make_async_copy(ego_hbm.at[pl.ds(0, _K)], rins[b],
                              gsems[b]).wait()

    def issue_scatter(slot, jj, b):
        pltpu.async_copy(routs[b], acc.at[row3.at[slot, jj]], ssems[b],
                         add=True)

    def wait_scatter(b):
        pltpu.make_async_copy(routs[b], acc.at[pl.ds(0, _K)],
                              ssems[b]).wait()

    def scale_buf(slot, j, b):
        rin = rins[b]
        rout = routs[b]

        def sg(g, carry):
            vv = val3[slot, j, pl.ds(16 * g, 16)]
            for jj in range(16):
                v = vv[jj]
                e = 16 * g + jj
                a = [rin[e, pl.ds(16 * d, 16)] for d in range(_DH // 16)]
                for d in range(_DH // 16):
                    rout[e, pl.ds(16 * d, 16)] = a[d] * v
            return carry

        lax.fori_loop(0, _K // 16, sg, 0)

    issue_gather(slot0, 0, 0)
    issue_gather(slot0, 1, 1)

    def super_group(k, carry):
        cur = lax.rem(k, 3)
        nxt = lax.rem(k + 1, 3)
        for j in range(_VG):
            i = _VG * k + j
            b = j % _NB

            if j == 4:
                @pl.when(k + 2 < _NG)
                def _():
                    issue_edata(k + 2)

            if j == 6:
                @pl.when(k + 1 < _NG)
                def _():
                    wait_edata()

            @pl.when(i + 2 < _NCHUNK)
            def _():
                if j < _VG - 2:
                    issue_gather(cur, j + 2, (j + 2) % _NB)
                else:
                    issue_gather(nxt, j - (_VG - 2), (j + 2) % _NB)

            wait_gather(b)

            @pl.when(i >= _NB)
            def _():
                wait_scatter(b)

            scale_buf(cur, j, b)
            issue_scatter(cur, j, b)
        return carry

    lax.fori_loop(0, _NG, super_group, 0)
    for b in range(_NB):
        wait_scatter(b)
    plsc.subcore_barrier()

    for j in range(_RPT // _ZR):
        r0 = pl.multiple_of(_RPT * s + _ZR * j, _ZR)
        ro = pl.multiple_of(c * _NPAD + _RPT * s + _ZR * j, _ZR)
        lrows = pl.ds(r0, _ZR)
        hrows = pl.ds(ro, _ZR)
        pltpu.sync_copy(acc.at[lrows], out_hbm.at[hrows])
        pltpu.sync_copy(acc.at[lrows], abuf)
        pltpu.sync_copy(sum_hbm.at[hrows], sbuf)

        def addrow(i, carry):
            for d in range(_DH // 16):
                sl = pl.ds(16 * d, 16)
                obuf[i, sl] = (sbuf[i, sl] + abuf[i, sl]) * scale
            return carry

        lax.fori_loop(0, _ZR, addrow, 0)
        pltpu.sync_copy(obuf, sout_hbm.at[hrows])


@functools.lru_cache(maxsize=None)
def _make_hop(scale):
    full = jax.ShapeDtypeStruct((2 * _NPAD, _DH), jnp.float32)
    return pl.kernel(
        functools.partial(_hop_body, scale),
        out_type=(full, full),
        mesh=plsc.VectorSubcoreMesh(core_axis_name="c", subcore_axis_name="s"),
        scratch_types=[
            pltpu.VMEM_SHARED((_NPAD, _DH), jnp.float32),
            pltpu.VMEM((3, _VG, _K), jnp.int32),
            pltpu.VMEM((3, _VG, _K), jnp.int32),
            pltpu.VMEM((3, _VG, _K), jnp.float32),
            pltpu.VMEM((_K, _DH), jnp.float32),
            pltpu.VMEM((_K, _DH), jnp.float32),
            pltpu.VMEM((_K, _DH), jnp.float32),
            pltpu.VMEM((_K, _DH), jnp.float32),
            pltpu.VMEM((_K, _DH), jnp.float32),
            pltpu.VMEM((_K, _DH), jnp.float32),
            pltpu.VMEM((_K, _DH), jnp.float32),
            pltpu.VMEM((_K, _DH), jnp.float32),
            pltpu.VMEM((_ZR, _DH), jnp.float32),
            pltpu.VMEM((_ZR, _DH), jnp.float32),
            pltpu.VMEM((_ZR, _DH), jnp.float32),
            pltpu.SemaphoreType.DMA,
            pltpu.SemaphoreType.DMA,
            pltpu.SemaphoreType.DMA,
            pltpu.SemaphoreType.DMA,
            pltpu.SemaphoreType.DMA,
            pltpu.SemaphoreType.DMA,
            pltpu.SemaphoreType.DMA,
            pltpu.SemaphoreType.DMA,
            pltpu.SemaphoreType.DMA,
        ],
        compiler_params=pltpu.CompilerParams(use_tc_tiling_on_sc=False),
        name=f"gcn_hop_scale_{scale}",
    )


def kernel(user_embed, item_embed, edge_row, edge_col, edge_val):
    ego = jnp.concatenate([user_embed, item_embed], axis=0)
    pad = jnp.zeros((_NPAD - _N_NODES, _DH), jnp.float32)
    ego_ab = jnp.concatenate([ego[:, :_DH], pad, ego[:, _DH:], pad], axis=0)

    epad = _NEPAD - _N_EDGES
    row = jnp.concatenate([
        edge_row.astype(jnp.int32),
        jnp.full((epad,), _DUMMY_ROW, jnp.int32),
    ]).reshape(_NS * _NCHUNK, _K)
    col = jnp.concatenate([
        edge_col.astype(jnp.int32),
        jnp.zeros((epad,), jnp.int32),
    ])
    col0 = col.reshape(_NS * _NCHUNK, _K)
    col1 = (col + _NPAD).reshape(_NS * _NCHUNK, _K)
    val = jnp.concatenate([
        edge_val.astype(jnp.float32),
        jnp.zeros((epad,), jnp.float32),
    ]).reshape(_NS * _NCHUNK, _K)

    sum_ab = jnp.zeros((2 * _NPAD, _DH), jnp.float32)
    for h in range(_N_HOPS):
        scale = (1.0 / _N_HOPS) if h == _N_HOPS - 1 else 1.0
        hop = _make_hop(scale)
        ego_ab, sum_ab = hop(ego_ab, sum_ab, row, col0, col1, val)

    all_emb = jnp.concatenate(
        [sum_ab[:_N_NODES], sum_ab[_NPAD:_NPAD + _N_NODES]], axis=1)
    return all_emb[:_N_USERS], all_emb[_N_USERS:]


# speedup vs baseline: 4.9316x; 1.0087x over previous
import functools

import jax
import jax.numpy as jnp
from jax import lax
from jax.experimental import pallas as pl
from jax.experimental.pallas import tpu as pltpu
from jax.experimental.pallas import tpu_sc as plsc

_N_USERS = 5000
_N_ITEMS = 5000
_N_NODES = _N_USERS + _N_ITEMS
_N_EDGES = 320000
_D = 128
_DH = _D // 2
_N_HOPS = 3

_NS = 16
_K = 128
_NCHUNK = 160
_EPT = _NCHUNK * _K
_NEPAD = _NS * _EPT
_NPAD = 10240
_RPT = _NPAD // _NS
_ZR = 32
_DUMMY_ROW = _N_NODES + 100
_NB = 4
_VG = 8


def _hop_body(scale, ego_hbm, sum_hbm, row_hbm, col0_hbm, col1_hbm, val_hbm,
              out_hbm, sout_hbm,
              acc, col3, row3, val3, ri0, ri1, ri2, ri3, ro0, ro1, ro2, ro3,
              ab0, ab1, sb0, sb1, ob0, ob1,
              g0, g1, g2, g3, s0, s1, s2, s3, esem,
              l0, l1, w0, w1, osem):
    c = lax.axis_index("c")
    s = lax.axis_index("s")
    rins = (ri0, ri1, ri2, ri3)
    routs = (ro0, ro1, ro2, ro3)
    gsems = (g0, g1, g2, g3)
    ssems = (s0, s1, s2, s3)
    abufs = (ab0, ab1)
    sbufs = (sb0, sb1)
    obufs = (ob0, ob1)
    lsems = (l0, l1)
    wsems = (w0, w1)
    _NG = _NCHUNK // _VG

    def edata_rows(g):
        eb = pl.multiple_of(s * _NCHUNK + _VG * g, _VG)
        return pl.ds(eb, _VG)

    def issue_edata(g):
        slot = lax.rem(g, 3)
        rows = edata_rows(g)

        @pl.when(c == 0)
        def _():
            pltpu.async_copy(col0_hbm.at[rows], col3.at[slot], esem)

        @pl.when(c == 1)
        def _():
            pltpu.async_copy(col1_hbm.at[rows], col3.at[slot], esem)

        pltpu.async_copy(row_hbm.at[rows], row3.at[slot], esem)
        pltpu.async_copy(val_hbm.at[rows], val3.at[slot], esem)

    def wait_edata():
        pltpu.make_async_copy(row_hbm.at[edata_rows(0)], col3.at[0],
                              esem).wait()
        pltpu.make_async_copy(row_hbm.at[edata_rows(0)], row3.at[0],
                              esem).wait()
        pltpu.make_async_copy(val_hbm.at[edata_rows(0)], val3.at[0],
                              esem).wait()

    slot0 = lax.rem(0 * s, 3)

    @pl.when(c == 0)
    def _():
        pltpu.sync_copy(col0_hbm.at[edata_rows(0)], col3.at[slot0])

    @pl.when(c == 1)
    def _():
        pltpu.sync_copy(col1_hbm.at[edata_rows(0)], col3.at[slot0])

    pltpu.sync_copy(row_hbm.at[edata_rows(0)], row3.at[slot0])
    pltpu.sync_copy(val_hbm.at[edata_rows(0)], val3.at[slot0])
    issue_edata(1)

    def zrow(i, carry):
        for d in range(_DH // 16):
            ab0[i, pl.ds(16 * d, 16)] = jnp.zeros((16,), jnp.float32)
        return carry

    lax.fori_loop(0, _ZR, zrow, 0)
    for j in range(_RPT // _ZR):
        r0 = pl.multiple_of(_RPT * s + _ZR * j, _ZR)
        pltpu.async_copy(ab0, acc.at[pl.ds(r0, _ZR)], g0)
        if j >= 4:
            pltpu.make_async_copy(ab0, acc.at[pl.ds(0, _ZR)], g0).wait()
    for j in range(4):
        pltpu.make_async_copy(ab0, acc.at[pl.ds(0, _ZR)], g0).wait()
    plsc.subcore_barrier()

    def issue_gather(slot, jj, b):
        pltpu.async_copy(ego_hbm.at[col3.at[slot, jj]], rins[b], gsems[b])

    def wait_gather(b):
        pltpu.make_async_copy(ego_hbm.at[pl.ds(0, _K)], rins[b],
                              gsems[b]).wait()

    def issue_scatter(slot, jj, b):
        pltpu.async_copy(routs[b], acc.at[row3.at[slot, jj]], ssems[b],
                         add=True)

    def wait_scatter(b):
        pltpu.make_async_copy(routs[b], acc.at[pl.ds(0, _K)],
                              ssems[b]).wait()

    def scale_buf(slot, j, b):
        rin = rins[b]
        rout = routs[b]

        def sg(g, carry):
            vv = val3[slot, j, pl.ds(16 * g, 16)]
            for jj in range(16):
                v = vv[jj]
                e = 16 * g + jj
                a = [rin[e, pl.ds(16 * d, 16)] for d in range(_DH // 16)]
                for d in range(_DH // 16):
                    rout[e, pl.ds(16 * d, 16)] = a[d] * v
            return carry

        lax.fori_loop(0, _K // 16, sg, 0)

    issue_gather(slot0, 0, 0)
    issue_gather(slot0, 1, 1)

    def super_group(k, carry):
        cur = lax.rem(k, 3)
        nxt = lax.rem(k + 1, 3)
        for j in range(_VG):
            i = _VG * k + j
            b = j % _NB

            if j == 4:
                @pl.when(k + 2 < _NG)
                def _():
                    issue_edata(k + 2)

            if j == 6:
                @pl.when(k + 1 < _NG)
                def _():
                    wait_edata()

            @pl.when(i + 2 < _NCHUNK)
            def _():
                if j < _VG - 2:
                    issue_gather(cur, j + 2, (j + 2) % _NB)
                else:
                    issue_gather(nxt, j - (_VG - 2), (j + 2) % _NB)

            wait_gather(b)

            @pl.when(i >= _NB)
            def _():
                wait_scatter(b)

            scale_buf(cur, j, b)
            issue_scatter(cur, j, b)
        return carry

    lax.fori_loop(0, _NG, super_group, 0)
    for b in range(_NB):
        wait_scatter(b)
    plsc.subcore_barrier()

    _NC2 = _RPT // _ZR

    def lrows(j):
        return pl.ds(pl.multiple_of(_RPT * s + _ZR * j, _ZR), _ZR)

    def hrows(j):
        return pl.ds(pl.multiple_of(c * _NPAD + _RPT * s + _ZR * j, _ZR), _ZR)

    def issue_loads(j):
        p = j % 2
        pltpu.async_copy(acc.at[lrows(j)], abufs[p], lsems[p])
        pltpu.async_copy(sum_hbm.at[hrows(j)], sbufs[p], wsems[p])

    def wait_loads(p):
        pltpu.make_async_copy(acc.at[pl.ds(0, _ZR)], abufs[p],
                              lsems[p]).wait()
        pltpu.make_async_copy(sum_hbm.at[pl.ds(0, _ZR)], sbufs[p],
                              wsems[p]).wait()

    issue_loads(0)
    for j in range(_NC2):
        p = j % 2
        if j + 1 < _NC2:
            issue_loads(j + 1)
        pltpu.sync_copy(acc.at[lrows(j)], out_hbm.at[hrows(j)])
        wait_loads(p)

        def addrow(i, carry, p=p):
            for d in range(_DH // 16):
                sl = pl.ds(16 * d, 16)
                obufs[p][i, sl] = (sbufs[p][i, sl] + abufs[p][i, sl]) * scale
            return carry

        lax.fori_loop(0, _ZR, addrow, 0)
        pltpu.sync_copy(obufs[p], sout_hbm.at[hrows(j)])


@functools.lru_cache(maxsize=None)
def _make_hop(scale):
    full = jax.ShapeDtypeStruct((2 * _NPAD, _DH), jnp.float32)
    return pl.kernel(
        functools.partial(_hop_body, scale),
        out_type=(full, full),
        mesh=plsc.VectorSubcoreMesh(core_axis_name="c", subcore_axis_name="s"),
        scratch_types=[
            pltpu.VMEM_SHARED((_NPAD, _DH), jnp.float32),
            pltpu.VMEM((3, _VG, _K), jnp.int32),
            pltpu.VMEM((3, _VG, _K), jnp.int32),
            pltpu.VMEM((3, _VG, _K), jnp.float32),
            pltpu.VMEM((_K, _DH), jnp.float32),
            pltpu.VMEM((_K, _DH), jnp.float32),
            pltpu.VMEM((_K, _DH), jnp.float32),
            pltpu.VMEM((_K, _DH), jnp.float32),
            pltpu.VMEM((_K, _DH), jnp.float32),
            pltpu.VMEM((_K, _DH), jnp.float32),
            pltpu.VMEM((_K, _DH), jnp.float32),
            pltpu.VMEM((_K, _DH), jnp.float32),
            pltpu.VMEM((_ZR, _DH), jnp.float32),
            pltpu.VMEM((_ZR, _DH), jnp.float32),
            pltpu.VMEM((_ZR, _DH), jnp.float32),
            pltpu.VMEM((_ZR, _DH), jnp.float32),
            pltpu.VMEM((_ZR, _DH), jnp.float32),
            pltpu.VMEM((_ZR, _DH), jnp.float32),
            pltpu.SemaphoreType.DMA,
            pltpu.SemaphoreType.DMA,
            pltpu.SemaphoreType.DMA,
            pltpu.SemaphoreType.DMA,
            pltpu.SemaphoreType.DMA,
            pltpu.SemaphoreType.DMA,
            pltpu.SemaphoreType.DMA,
            pltpu.SemaphoreType.DMA,
            pltpu.SemaphoreType.DMA,
            pltpu.SemaphoreType.DMA,
            pltpu.SemaphoreType.DMA,
            pltpu.SemaphoreType.DMA,
            pltpu.SemaphoreType.DMA,
            pltpu.SemaphoreType.DMA,
        ],
        compiler_params=pltpu.CompilerParams(use_tc_tiling_on_sc=False),
        name=f"gcn_hop_scale_{scale}",
    )


def kernel(user_embed, item_embed, edge_row, edge_col, edge_val):
    ego = jnp.concatenate([user_embed, item_embed], axis=0)
    pad = jnp.zeros((_NPAD - _N_NODES, _DH), jnp.float32)
    ego_ab = jnp.concatenate([ego[:, :_DH], pad, ego[:, _DH:], pad], axis=0)

    epad = _NEPAD - _N_EDGES
    row = jnp.concatenate([
        edge_row.astype(jnp.int32),
        jnp.full((epad,), _DUMMY_ROW, jnp.int32),
    ]).reshape(_NS * _NCHUNK, _K)
    col = jnp.concatenate([
        edge_col.astype(jnp.int32),
        jnp.zeros((epad,), jnp.int32),
    ])
    col0 = col.reshape(_NS * _NCHUNK, _K)
    col1 = (col + _NPAD).reshape(_NS * _NCHUNK, _K)
    val = jnp.concatenate([
        edge_val.astype(jnp.float32),
        jnp.zeros((epad,), jnp.float32),
    ]).reshape(_NS * _NCHUNK, _K)

    sum_ab = jnp.zeros((2 * _NPAD, _DH), jnp.float32)
    for h in range(_N_HOPS):
        scale = (1.0 / _N_HOPS) if h == _N_HOPS - 1 else 1.0
        hop = _make_hop(scale)
        ego_ab, sum_ab = hop(ego_ab, sum_ab, row, col0, col1, val)

    all_emb = jnp.concatenate(
        [sum_ab[:_N_NODES], sum_ab[_NPAD:_NPAD + _N_NODES]], axis=1)
    return all_emb[:_N_USERS], all_emb[_N_USERS:]
